# baseline (speedup 1.0000x reference)
import functools

import jax
import jax.numpy as jnp
from jax import lax
from jax.experimental import pallas as pl
from jax.experimental.pallas import tpu as pltpu
from jax.experimental.pallas import tpu_sc as plsc

_CHUNK_IDX = 128
_LANES = 16


def _gather_kernel(B, V, D, NW, b_per_w, n_idx_chunks):
    mesh = plsc.VectorSubcoreMesh(core_axis_name="c", subcore_axis_name="s")
    n_groups = b_per_w // _LANES

    @functools.partial(
        pl.kernel,
        mesh=mesh,
        out_type=jax.ShapeDtypeStruct((B, D), jnp.float32),
        scratch_types=[
            pltpu.VMEM((n_idx_chunks, _CHUNK_IDX), jnp.int32),
            pltpu.VMEM((n_idx_chunks, _CHUNK_IDX), jnp.int32),
            pltpu.VMEM((b_per_w // 2, 2 * D), jnp.float32),
            pltpu.VMEM((b_per_w // 2, D), jnp.float32),
            pltpu.SemaphoreType.DMA,
        ],
    )
    def k(idx_hbm, lin_hbm, out_hbm, idx_v, p_v, rows_v, out_v, sem):
        nc = plsc.get_sparse_core_info().num_cores
        wid = lax.axis_index("s") * nc + lax.axis_index("c")
        row_base = wid * n_idx_chunks
        pltpu.sync_copy(idx_hbm.at[pl.ds(row_base, n_idx_chunks)], idx_v)

        per_row = _CHUNK_IDX // _LANES

        def shift_group(g, carry):
            r = g // per_row
            col = (g % per_row) * _LANES
            vec = idx_v[r, pl.ds(col, _LANES)]
            p_v[r, pl.ds(col, _LANES)] = lax.shift_right_logical(vec, 1)
            return carry

        lax.fori_loop(0, n_groups, shift_group, 0)

        half_chunks = n_idx_chunks // 2
        groups_per_half = n_groups // 2
        for hh in range(2):
            copies = []
            for j in range(half_chunks):
                copies.append(
                    pltpu.async_copy(
                        lin_hbm.at[p_v.at[hh * half_chunks + j]],
                        rows_v.at[pl.ds(j * _CHUNK_IDX, _CHUNK_IDX)],
                        sem,
                    )
                )
            for c in copies:
                c.wait()

            def select_group(g, carry, hh=hh):
                gg = hh * groups_per_half + g
                r = gg // per_row
                col = (gg % per_row) * _LANES
                vec = idx_v[r, pl.ds(col, _LANES)]
                for j in range(_LANES):
                    i = g * _LANES + j
                    src = (vec[j] & 1) * D
                    for kk in range(D // _LANES):
                        out_v[i, pl.ds(kk * _LANES, _LANES)] = rows_v[
                            i, pl.ds(src + kk * _LANES, _LANES)
                        ]
                return carry

            lax.fori_loop(0, groups_per_half, select_group, 0)
            pltpu.sync_copy(
                out_v,
                out_hbm.at[pl.ds(wid * b_per_w + hh * (b_per_w // 2), b_per_w // 2)],
            )

    return k


def kernel(indices, autodecoder_embeddings):
    (B,) = indices.shape
    V, D = autodecoder_embeddings.shape
    info = plsc.get_sparse_core_info()
    NC, NS = info.num_cores, info.num_subcores
    NW = NC * NS
    b_per_w = B // NW
    n_idx_chunks = b_per_w // _CHUNK_IDX
    idx2d = indices.astype(jnp.int32).reshape(NW * n_idx_chunks, _CHUNK_IDX)
    lin = autodecoder_embeddings.reshape(V // 2, 2 * D)
    return _gather_kernel(B, V, D, NW, b_per_w, n_idx_chunks)(idx2d, lin)
